# baseline (speedup 1.0000x reference)
import functools

import jax
import jax.numpy as jnp
from jax import lax
from jax.experimental import pallas as pl
from jax.experimental.pallas import tpu as pltpu
from jax.experimental.pallas import tpu_sc as plsc

EMB = 64
HID = 64
NC, NS = 2, 16
NW = NC * NS
CHUNK = 128
PBLK = 4096
LANES = 16


def _tc_project_table(table_t, W):
    V = table_t.shape[1]
    grid = (V + PBLK - 1) // PBLK

    def body(et_ref, w_ref, p_ref):
        y = lax.dot_general(et_ref[...], w_ref[...], (((0,), (1,)), ((), ())),
                            preferred_element_type=jnp.float32)
        p_ref[...] = jnp.concatenate(
            [y, jnp.zeros((PBLK, 128 - HID), jnp.float32)], axis=1)

    return pl.pallas_call(
        body,
        grid=(grid,),
        in_specs=[
            pl.BlockSpec((EMB, PBLK), lambda i: (0, i)),
            pl.BlockSpec((HID, EMB), lambda i: (0, 0)),
        ],
        out_specs=pl.BlockSpec((PBLK, 128), lambda i: (i, 0)),
        out_shape=jax.ShapeDtypeStruct((V, 128), jnp.float32),
    )(table_t, W)


def _sc_gather_transpose(idx1, idx2, table, batch, seq):
    hw = NW // 2
    cpw = idx1.shape[1]
    bpc = batch // CHUNK

    mesh = plsc.VectorSubcoreMesh(core_axis_name="c", subcore_axis_name="s")

    @functools.partial(
        pl.kernel,
        out_type=(jax.ShapeDtypeStruct((seq, HID, batch), jnp.float32),
                  jax.ShapeDtypeStruct((seq, HID, batch), jnp.float32)),
        mesh=mesh,
        scratch_types=[
            pltpu.VMEM((cpw, CHUNK), jnp.int32),
            pltpu.VMEM((2, CHUNK, 128), jnp.float32),
            pltpu.VMEM((2, HID, CHUNK), jnp.float32),
            pltpu.SemaphoreType.DMA,
            pltpu.SemaphoreType.DMA,
            pltpu.SemaphoreType.DMA,
            pltpu.SemaphoreType.DMA,
        ],
        compiler_params=pltpu.CompilerParams(needs_layout_passes=False),
    )
    def k(idx1_hbm, idx2_hbm, table_hbm, out1_hbm, out2_hbm,
          idx_v, rows_v, tbuf, gs0, gs1, os0, os1):
        wid = lax.axis_index("s") * NC + lax.axis_index("c")
        sid = wid // hw
        ww = wid % hw
        gsems = (gs0, gs1)
        osems = (os0, os1)

        def run(idx_hbm, out_hbm):
            pltpu.sync_copy(idx_hbm.at[ww], idx_v)

            def gather(g, b):
                return pltpu.make_async_copy(
                    table_hbm.at[idx_v.at[g]], rows_v.at[b], gsems[b])

            def scatter(g, b):
                c = ww * cpw + g
                s = c // bpc
                b0 = (c % bpc) * CHUNK
                return pltpu.make_async_copy(
                    tbuf.at[b],
                    out_hbm.at[s, :, pl.ds(b0, CHUNK)],
                    osems[b])

            def transpose_chunk(b):
                for grp in range(CHUNK // LANES):
                    rowidx = lax.iota(jnp.int32, LANES) + grp * LANES

                    @pl.loop(0, HID, unroll=16)
                    def _(h):
                        hvec = jnp.zeros((LANES,), jnp.int32) + h
                        vals = plsc.load_gather(rows_v.at[b], [rowidx, hvec])
                        plsc.store_scatter(tbuf.at[b], [hvec, rowidx], vals)

            gather(0, 0).start()

            @pl.loop(0, cpw, step=2)
            def _(i):
                for b in (0, 1):
                    g = i + b
                    @pl.when(g + 1 < cpw)
                    def _():
                        gather(g + 1, 1 - b).start()

                    gather(g, b).wait()
                    @pl.when(g >= 2)
                    def _():
                        scatter(g - 2, b).wait()
                    transpose_chunk(b)
                    scatter(g, b).start()

            scatter(cpw - 2, 0).wait()
            scatter(cpw - 1, 1).wait()

        @pl.when(sid == 0)
        def _():
            run(idx1_hbm, out1_hbm)

        @pl.when(sid == 1)
        def _():
            run(idx2_hbm, out2_hbm)

    return k(idx1, idx2, table)


def kernel(sent1, sent2, embedding, W):
    batch, seq = sent1.shape
    hw = NW // 2
    proj = _tc_project_table(embedding.T, W)
    idx1 = sent1.T.reshape(hw, -1, CHUNK).astype(jnp.int32)
    idx2 = sent2.T.reshape(hw, -1, CHUNK).astype(jnp.int32)
    t1, t2 = _sc_gather_transpose(idx1, idx2, proj, batch, seq)
    return (jnp.transpose(t1, (2, 0, 1)), jnp.transpose(t2, (2, 0, 1)))


# speedup vs baseline: 1.5606x; 1.5606x over previous
import functools

import jax
import jax.numpy as jnp
from jax import lax
from jax.experimental import pallas as pl
from jax.experimental.pallas import tpu as pltpu
from jax.experimental.pallas import tpu_sc as plsc

EMB = 64
HID = 64
NC, NS = 2, 16
NW = NC * NS
CHUNK = 128
PBLK = 2048
PGRID = 245
SPLIT = PGRID * PBLK


def _tc_project_table(table_t, W):
    V = table_t.shape[1]
    last_hi = (V - 1) // PBLK

    def body(lo_ref, hi_ref, w_ref, p_ref):
        dn = (((0,), (1,)), ((), ()))
        w = w_ref[...]
        ylo = lax.dot_general(lo_ref[...], w, dn,
                              preferred_element_type=jnp.float32)
        yhi = lax.dot_general(hi_ref[...], w, dn,
                              preferred_element_type=jnp.float32)
        p_ref[...] = jnp.concatenate([ylo, yhi], axis=1)

    return pl.pallas_call(
        body,
        grid=(PGRID,),
        in_specs=[
            pl.BlockSpec((EMB, PBLK), lambda i: (0, i)),
            pl.BlockSpec((EMB, PBLK),
                         lambda i: (0, jnp.minimum(i + PGRID, last_hi))),
            pl.BlockSpec((HID, EMB), lambda i: (0, 0)),
        ],
        out_specs=pl.BlockSpec((PBLK, 2 * HID), lambda i: (i, 0)),
        out_shape=jax.ShapeDtypeStruct((SPLIT, 2 * HID), jnp.float32),
    )(table_t, table_t, W)


def _sc_gather(idx1, idx2, table):
    hw = NW // 2
    cpw = idx1.shape[1]
    half_rows = hw * cpw * CHUNK

    mesh = plsc.VectorSubcoreMesh(core_axis_name="c", subcore_axis_name="s")

    @functools.partial(
        pl.kernel,
        out_type=(jax.ShapeDtypeStruct((half_rows, 128), jnp.float32),
                  jax.ShapeDtypeStruct((half_rows, 128), jnp.float32)),
        mesh=mesh,
        scratch_types=[
            pltpu.VMEM((cpw, CHUNK), jnp.int32),
            pltpu.VMEM((2, CHUNK, 128), jnp.float32),
            pltpu.SemaphoreType.DMA,
            pltpu.SemaphoreType.DMA,
            pltpu.SemaphoreType.DMA,
            pltpu.SemaphoreType.DMA,
        ],
    )
    def k(idx1_hbm, idx2_hbm, table_hbm, out1_hbm, out2_hbm,
          idx_v, rows_v, gs0, gs1, os0, os1):
        wid = lax.axis_index("s") * NC + lax.axis_index("c")
        sid = wid // hw
        ww = wid % hw
        gsems = (gs0, gs1)
        osems = (os0, os1)

        def run(idx_hbm, out_hbm):
            pltpu.sync_copy(idx_hbm.at[ww], idx_v)

            def gather(g, b):
                return pltpu.make_async_copy(
                    table_hbm.at[idx_v.at[g]], rows_v.at[b], gsems[b])

            def scatter(g, b):
                return pltpu.make_async_copy(
                    rows_v.at[b],
                    out_hbm.at[pl.ds((ww * cpw + g) * CHUNK, CHUNK)],
                    osems[b])

            gather(0, 0).start()

            @pl.loop(0, cpw, step=2)
            def _(i):
                for b in (0, 1):
                    g = i + b
                    @pl.when(g + 1 < cpw)
                    def _():
                        @pl.when(g >= 1)
                        def _():
                            scatter(g - 1, 1 - b).wait()
                        gather(g + 1, 1 - b).start()

                    gather(g, b).wait()
                    scatter(g, b).start()

            scatter(cpw - 2, 0).wait()
            scatter(cpw - 1, 1).wait()

        @pl.when(sid == 0)
        def _():
            run(idx1_hbm, out1_hbm)

        @pl.when(sid == 1)
        def _():
            run(idx2_hbm, out2_hbm)

    return k(idx1, idx2, table)


def _tc_finalize(g, par, batch, seq):
    def body(x_ref, par_ref, o_ref):
        xt = jnp.transpose(x_ref[...])
        p = par_ref[0]
        o_ref[...] = jnp.where(p == 1, xt[HID:], xt[:HID])[None]

    return pl.pallas_call(
        body,
        grid=(seq,),
        in_specs=[
            pl.BlockSpec((batch, 128), lambda i: (i, 0)),
            pl.BlockSpec((1, 1, batch), lambda i: (i, 0, 0)),
        ],
        out_specs=pl.BlockSpec((1, HID, batch), lambda i: (i, 0, 0)),
        out_shape=jax.ShapeDtypeStruct((seq, HID, batch), jnp.float32),
    )(g, par)


def kernel(sent1, sent2, embedding, W):
    batch, seq = sent1.shape
    hw = NW // 2
    proj = _tc_project_table(embedding.T, W)
    s1t = sent1.T.astype(jnp.int32)
    s2t = sent2.T.astype(jnp.int32)
    idx1 = jnp.where(s1t >= SPLIT, s1t - SPLIT, s1t).reshape(hw, -1, CHUNK)
    idx2 = jnp.where(s2t >= SPLIT, s2t - SPLIT, s2t).reshape(hw, -1, CHUNK)
    par1 = (s1t >= SPLIT).astype(jnp.int32).reshape(seq, 1, batch)
    par2 = (s2t >= SPLIT).astype(jnp.int32).reshape(seq, 1, batch)
    g1, g2 = _sc_gather(idx1, idx2, proj)
    t1 = _tc_finalize(g1, par1, batch, seq)
    t2 = _tc_finalize(g2, par2, batch, seq)
    return (jnp.transpose(t1, (2, 0, 1)), jnp.transpose(t2, (2, 0, 1)))


# speedup vs baseline: 1.7507x; 1.1218x over previous
import functools

import jax
import jax.numpy as jnp
from jax import lax
from jax.experimental import pallas as pl
from jax.experimental.pallas import tpu as pltpu
from jax.experimental.pallas import tpu_sc as plsc

EMB = 64
HID = 64
NC, NS = 2, 16
NW = NC * NS
CHUNK = 128
PBLK = 4096
PGRID = 123
SPLIT = PGRID * PBLK


def _tc_project_table(table_t, W):
    V = table_t.shape[1]
    last_hi = (V - 1) // PBLK

    def body(lo_ref, hi_ref, w_ref, p_ref):
        dn = (((0,), (1,)), ((), ()))
        w = w_ref[...]
        ylo = lax.dot_general(lo_ref[...], w, dn,
                              preferred_element_type=jnp.float32)
        yhi = lax.dot_general(hi_ref[...], w, dn,
                              preferred_element_type=jnp.float32)
        p_ref[...] = jnp.concatenate([ylo, yhi], axis=1)

    return pl.pallas_call(
        body,
        grid=(PGRID,),
        in_specs=[
            pl.BlockSpec((EMB, PBLK), lambda i: (0, i)),
            pl.BlockSpec((EMB, PBLK),
                         lambda i: (0, jnp.minimum(i + PGRID, last_hi))),
            pl.BlockSpec((HID, EMB), lambda i: (0, 0)),
        ],
        out_specs=pl.BlockSpec((PBLK, 2 * HID), lambda i: (i, 0)),
        out_shape=jax.ShapeDtypeStruct((SPLIT, 2 * HID), jnp.float32),
    )(table_t, table_t, W)


def _sc_gather(idx1, idx2, table):
    hw = NW // 2
    cpw = idx1.shape[1]
    half_rows = hw * cpw * CHUNK

    mesh = plsc.VectorSubcoreMesh(core_axis_name="c", subcore_axis_name="s")

    @functools.partial(
        pl.kernel,
        out_type=(jax.ShapeDtypeStruct((half_rows, 128), jnp.float32),
                  jax.ShapeDtypeStruct((half_rows, 128), jnp.float32)),
        mesh=mesh,
        scratch_types=[
            pltpu.VMEM((cpw, CHUNK), jnp.int32),
            pltpu.VMEM((2, CHUNK, 128), jnp.float32),
            pltpu.SemaphoreType.DMA,
            pltpu.SemaphoreType.DMA,
            pltpu.SemaphoreType.DMA,
            pltpu.SemaphoreType.DMA,
        ],
    )
    def k(idx1_hbm, idx2_hbm, table_hbm, out1_hbm, out2_hbm,
          idx_v, rows_v, gs0, gs1, os0, os1):
        wid = lax.axis_index("s") * NC + lax.axis_index("c")
        sid = wid // hw
        ww = wid % hw
        gsems = (gs0, gs1)
        osems = (os0, os1)

        def run(idx_hbm, out_hbm):
            pltpu.sync_copy(idx_hbm.at[ww], idx_v)

            def gather(g, b):
                return pltpu.make_async_copy(
                    table_hbm.at[idx_v.at[g]], rows_v.at[b], gsems[b])

            def scatter(g, b):
                return pltpu.make_async_copy(
                    rows_v.at[b],
                    out_hbm.at[pl.ds((ww * cpw + g) * CHUNK, CHUNK)],
                    osems[b])

            gather(0, 0).start()

            @pl.loop(0, cpw, step=2)
            def _(i):
                for b in (0, 1):
                    g = i + b
                    @pl.when(g + 1 < cpw)
                    def _():
                        @pl.when(g >= 1)
                        def _():
                            scatter(g - 1, 1 - b).wait()
                        gather(g + 1, 1 - b).start()

                    gather(g, b).wait()
                    scatter(g, b).start()

            scatter(cpw - 2, 0).wait()
            scatter(cpw - 1, 1).wait()

        @pl.when(sid == 0)
        def _():
            run(idx1_hbm, out1_hbm)

        @pl.when(sid == 1)
        def _():
            run(idx2_hbm, out2_hbm)

    return k(idx1, idx2, table)


def _tc_finalize(g, par, batch, seq):
    def body(x_ref, par_ref, o_ref):
        xt = jnp.transpose(x_ref[...])
        p = par_ref[0]
        o_ref[...] = jnp.where(p == 1, xt[HID:], xt[:HID])[None]

    return pl.pallas_call(
        body,
        grid=(seq,),
        in_specs=[
            pl.BlockSpec((batch, 128), lambda i: (i, 0)),
            pl.BlockSpec((1, 1, batch), lambda i: (i, 0, 0)),
        ],
        out_specs=pl.BlockSpec((1, HID, batch), lambda i: (i, 0, 0)),
        out_shape=jax.ShapeDtypeStruct((seq, HID, batch), jnp.float32),
    )(g, par)


def kernel(sent1, sent2, embedding, W):
    batch, seq = sent1.shape
    hw = NW // 2
    proj = _tc_project_table(embedding.T, W)
    s1t = sent1.T.astype(jnp.int32)
    s2t = sent2.T.astype(jnp.int32)
    idx1 = jnp.where(s1t >= SPLIT, s1t - SPLIT, s1t).reshape(hw, -1, CHUNK)
    idx2 = jnp.where(s2t >= SPLIT, s2t - SPLIT, s2t).reshape(hw, -1, CHUNK)
    par1 = (s1t >= SPLIT).astype(jnp.int32).reshape(seq, 1, batch)
    par2 = (s2t >= SPLIT).astype(jnp.int32).reshape(seq, 1, batch)
    g1, g2 = _sc_gather(idx1, idx2, proj)
    t1 = _tc_finalize(g1, par1, batch, seq)
    t2 = _tc_finalize(g2, par2, batch, seq)
    return (jnp.transpose(t1, (2, 0, 1)), jnp.transpose(t2, (2, 0, 1)))


# speedup vs baseline: 1.8593x; 1.0621x over previous
import functools

import jax
import jax.numpy as jnp
from jax import lax
from jax.experimental import pallas as pl
from jax.experimental.pallas import tpu as pltpu
from jax.experimental.pallas import tpu_sc as plsc

EMB = 64
HID = 64
NC, NS = 2, 16
NW = NC * NS
CHUNK = 128
PBLK = 8192
PGRID = 62
SPLIT = PGRID * PBLK


def _tc_project_table(table_t, W):
    V = table_t.shape[1]
    last_hi = (V - 1) // PBLK

    def body(lo_ref, hi_ref, w_ref, p_ref):
        dn = (((0,), (1,)), ((), ()))
        w = w_ref[...]
        ylo = lax.dot_general(lo_ref[...], w, dn,
                              preferred_element_type=jnp.float32)
        yhi = lax.dot_general(hi_ref[...], w, dn,
                              preferred_element_type=jnp.float32)
        p_ref[...] = jnp.concatenate([ylo, yhi], axis=1)

    return pl.pallas_call(
        body,
        grid=(PGRID,),
        in_specs=[
            pl.BlockSpec((EMB, PBLK), lambda i: (0, i)),
            pl.BlockSpec((EMB, PBLK),
                         lambda i: (0, jnp.minimum(i + PGRID, last_hi))),
            pl.BlockSpec((HID, EMB), lambda i: (0, 0)),
        ],
        out_specs=pl.BlockSpec((PBLK, 2 * HID), lambda i: (i, 0)),
        out_shape=jax.ShapeDtypeStruct((SPLIT, 2 * HID), jnp.float32),
    )(table_t, table_t, W)


def _sc_gather(idx1, idx2, table):
    hw = NW // 2
    cpw = idx1.shape[1]
    half_rows = hw * cpw * CHUNK

    mesh = plsc.VectorSubcoreMesh(core_axis_name="c", subcore_axis_name="s")

    @functools.partial(
        pl.kernel,
        out_type=(jax.ShapeDtypeStruct((half_rows, 128), jnp.float32),
                  jax.ShapeDtypeStruct((half_rows, 128), jnp.float32)),
        mesh=mesh,
        scratch_types=[
            pltpu.VMEM((cpw, CHUNK), jnp.int32),
            pltpu.VMEM((2, CHUNK, 128), jnp.float32),
            pltpu.SemaphoreType.DMA,
            pltpu.SemaphoreType.DMA,
            pltpu.SemaphoreType.DMA,
            pltpu.SemaphoreType.DMA,
        ],
    )
    def k(idx1_hbm, idx2_hbm, table_hbm, out1_hbm, out2_hbm,
          idx_v, rows_v, gs0, gs1, os0, os1):
        wid = lax.axis_index("s") * NC + lax.axis_index("c")
        sid = wid // hw
        ww = wid % hw
        gsems = (gs0, gs1)
        osems = (os0, os1)

        def run(idx_hbm, out_hbm):
            pltpu.sync_copy(idx_hbm.at[ww], idx_v)

            def gather(g, b):
                return pltpu.make_async_copy(
                    table_hbm.at[idx_v.at[g]], rows_v.at[b], gsems[b])

            def scatter(g, b):
                return pltpu.make_async_copy(
                    rows_v.at[b],
                    out_hbm.at[pl.ds((ww * cpw + g) * CHUNK, CHUNK)],
                    osems[b])

            gather(0, 0).start()

            @pl.loop(0, cpw, step=2)
            def _(i):
                for b in (0, 1):
                    g = i + b
                    @pl.when(g + 1 < cpw)
                    def _():
                        @pl.when(g >= 1)
                        def _():
                            scatter(g - 1, 1 - b).wait()
                        gather(g + 1, 1 - b).start()

                    gather(g, b).wait()
                    scatter(g, b).start()

            scatter(cpw - 2, 0).wait()
            scatter(cpw - 1, 1).wait()

        @pl.when(sid == 0)
        def _():
            run(idx1_hbm, out1_hbm)

        @pl.when(sid == 1)
        def _():
            run(idx2_hbm, out2_hbm)

    return k(idx1, idx2, table)


def _tc_finalize(g, par, batch, seq):
    def body(x_ref, par_ref, o_ref):
        xt = jnp.transpose(x_ref[...])
        p = par_ref[0]
        o_ref[...] = jnp.where(p == 1, xt[HID:], xt[:HID])[None]

    return pl.pallas_call(
        body,
        grid=(seq,),
        in_specs=[
            pl.BlockSpec((batch, 128), lambda i: (i, 0)),
            pl.BlockSpec((1, 1, batch), lambda i: (i, 0, 0)),
        ],
        out_specs=pl.BlockSpec((1, HID, batch), lambda i: (i, 0, 0)),
        out_shape=jax.ShapeDtypeStruct((seq, HID, batch), jnp.float32),
    )(g, par)


def kernel(sent1, sent2, embedding, W):
    batch, seq = sent1.shape
    hw = NW // 2
    proj = _tc_project_table(embedding.T, W)
    s1t = sent1.T.astype(jnp.int32)
    s2t = sent2.T.astype(jnp.int32)
    idx1 = jnp.where(s1t >= SPLIT, s1t - SPLIT, s1t).reshape(hw, -1, CHUNK)
    idx2 = jnp.where(s2t >= SPLIT, s2t - SPLIT, s2t).reshape(hw, -1, CHUNK)
    par1 = (s1t >= SPLIT).astype(jnp.int32).reshape(seq, 1, batch)
    par2 = (s2t >= SPLIT).astype(jnp.int32).reshape(seq, 1, batch)
    g1, g2 = _sc_gather(idx1, idx2, proj)
    t1 = _tc_finalize(g1, par1, batch, seq)
    t2 = _tc_finalize(g2, par2, batch, seq)
    return (jnp.transpose(t1, (2, 0, 1)), jnp.transpose(t2, (2, 0, 1)))


# speedup vs baseline: 1.9085x; 1.0264x over previous
import functools

import jax
import jax.numpy as jnp
from jax import lax
from jax.experimental import pallas as pl
from jax.experimental.pallas import tpu as pltpu
from jax.experimental.pallas import tpu_sc as plsc

EMB = 64
HID = 64
NC, NS = 2, 16
NW = NC * NS
CHUNK = 128
PBLK = 16384
PGRID = 31
SPLIT = PGRID * PBLK


def _tc_project_table(table_t, W):
    V = table_t.shape[1]
    last_hi = (V - 1) // PBLK

    def body(lo_ref, hi_ref, w_ref, p_ref):
        dn = (((0,), (1,)), ((), ()))
        w = w_ref[...]
        ylo = lax.dot_general(lo_ref[...], w, dn,
                              preferred_element_type=jnp.float32)
        yhi = lax.dot_general(hi_ref[...], w, dn,
                              preferred_element_type=jnp.float32)
        p_ref[...] = jnp.concatenate([ylo, yhi], axis=1)

    return pl.pallas_call(
        body,
        grid=(PGRID,),
        in_specs=[
            pl.BlockSpec((EMB, PBLK), lambda i: (0, i)),
            pl.BlockSpec((EMB, PBLK),
                         lambda i: (0, jnp.minimum(i + PGRID, last_hi))),
            pl.BlockSpec((HID, EMB), lambda i: (0, 0)),
        ],
        out_specs=pl.BlockSpec((PBLK, 2 * HID), lambda i: (i, 0)),
        out_shape=jax.ShapeDtypeStruct((SPLIT, 2 * HID), jnp.float32),
    )(table_t, table_t, W)


def _sc_gather(idx1, idx2, table):
    hw = NW // 2
    cpw = idx1.shape[1]
    half_rows = hw * cpw * CHUNK

    mesh = plsc.VectorSubcoreMesh(core_axis_name="c", subcore_axis_name="s")

    @functools.partial(
        pl.kernel,
        out_type=(jax.ShapeDtypeStruct((half_rows, 128), jnp.float32),
                  jax.ShapeDtypeStruct((half_rows, 128), jnp.float32)),
        mesh=mesh,
        scratch_types=[
            pltpu.VMEM((cpw, CHUNK), jnp.int32),
            pltpu.VMEM((2, CHUNK, 128), jnp.float32),
            pltpu.SemaphoreType.DMA,
            pltpu.SemaphoreType.DMA,
            pltpu.SemaphoreType.DMA,
            pltpu.SemaphoreType.DMA,
        ],
    )
    def k(idx1_hbm, idx2_hbm, table_hbm, out1_hbm, out2_hbm,
          idx_v, rows_v, gs0, gs1, os0, os1):
        wid = lax.axis_index("s") * NC + lax.axis_index("c")
        sid = wid // hw
        ww = wid % hw
        gsems = (gs0, gs1)
        osems = (os0, os1)

        def run(idx_hbm, out_hbm):
            pltpu.sync_copy(idx_hbm.at[ww], idx_v)

            def gather(g, b):
                return pltpu.make_async_copy(
                    table_hbm.at[idx_v.at[g]], rows_v.at[b], gsems[b])

            def scatter(g, b):
                return pltpu.make_async_copy(
                    rows_v.at[b],
                    out_hbm.at[pl.ds((ww * cpw + g) * CHUNK, CHUNK)],
                    osems[b])

            gather(0, 0).start()

            @pl.loop(0, cpw, step=2)
            def _(i):
                for b in (0, 1):
                    g = i + b
                    @pl.when(g + 1 < cpw)
                    def _():
                        @pl.when(g >= 1)
                        def _():
                            scatter(g - 1, 1 - b).wait()
                        gather(g + 1, 1 - b).start()

                    gather(g, b).wait()
                    scatter(g, b).start()

            scatter(cpw - 2, 0).wait()
            scatter(cpw - 1, 1).wait()

        @pl.when(sid == 0)
        def _():
            run(idx1_hbm, out1_hbm)

        @pl.when(sid == 1)
        def _():
            run(idx2_hbm, out2_hbm)

    return k(idx1, idx2, table)


def _tc_finalize(g, par, batch, seq):
    def body(x_ref, par_ref, o_ref):
        xt = jnp.transpose(x_ref[...])
        p = par_ref[0]
        o_ref[...] = jnp.where(p == 1, xt[HID:], xt[:HID])[None]

    return pl.pallas_call(
        body,
        grid=(seq,),
        in_specs=[
            pl.BlockSpec((batch, 128), lambda i: (i, 0)),
            pl.BlockSpec((1, 1, batch), lambda i: (i, 0, 0)),
        ],
        out_specs=pl.BlockSpec((1, HID, batch), lambda i: (i, 0, 0)),
        out_shape=jax.ShapeDtypeStruct((seq, HID, batch), jnp.float32),
    )(g, par)


def kernel(sent1, sent2, embedding, W):
    batch, seq = sent1.shape
    hw = NW // 2
    proj = _tc_project_table(embedding.T, W)
    s1t = sent1.T.astype(jnp.int32)
    s2t = sent2.T.astype(jnp.int32)
    idx1 = jnp.where(s1t >= SPLIT, s1t - SPLIT, s1t).reshape(hw, -1, CHUNK)
    idx2 = jnp.where(s2t >= SPLIT, s2t - SPLIT, s2t).reshape(hw, -1, CHUNK)
    par1 = (s1t >= SPLIT).astype(jnp.int32).reshape(seq, 1, batch)
    par2 = (s2t >= SPLIT).astype(jnp.int32).reshape(seq, 1, batch)
    g1, g2 = _sc_gather(idx1, idx2, proj)
    t1 = _tc_finalize(g1, par1, batch, seq)
    t2 = _tc_finalize(g2, par2, batch, seq)
    return (jnp.transpose(t1, (2, 0, 1)), jnp.transpose(t2, (2, 0, 1)))


# speedup vs baseline: 1.9836x; 1.0393x over previous
import functools

import jax
import jax.numpy as jnp
from jax import lax
from jax.experimental import pallas as pl
from jax.experimental.pallas import tpu as pltpu
from jax.experimental.pallas import tpu_sc as plsc

EMB = 64
HID = 64
NC, NS = 2, 16
NW = NC * NS
CHUNK = 128
PBLK = 16384
PGRID = 31
SPLIT = PGRID * PBLK


def _tc_project_table(table_t, W):
    V = table_t.shape[1]
    last_hi = (V - 1) // PBLK

    def body(lo_ref, hi_ref, w_ref, p_ref):
        dn = (((0,), (1,)), ((), ()))
        w = w_ref[...]
        ylo = lax.dot_general(lo_ref[...], w, dn,
                              preferred_element_type=jnp.float32)
        yhi = lax.dot_general(hi_ref[...], w, dn,
                              preferred_element_type=jnp.float32)
        p_ref[...] = jnp.concatenate([ylo, yhi], axis=1)

    return pl.pallas_call(
        body,
        grid=(PGRID,),
        in_specs=[
            pl.BlockSpec((EMB, PBLK), lambda i: (0, i)),
            pl.BlockSpec((EMB, PBLK),
                         lambda i: (0, jnp.minimum(i + PGRID, last_hi))),
            pl.BlockSpec((HID, EMB), lambda i: (0, 0)),
        ],
        out_specs=pl.BlockSpec((PBLK, 2 * HID), lambda i: (i, 0)),
        out_shape=jax.ShapeDtypeStruct((SPLIT, 2 * HID), jnp.float32),
    )(table_t, table_t, W)


def _sc_gather(idx3d, table):
    cpw = idx3d.shape[1]
    n_rows = NW * cpw * CHUNK

    mesh = plsc.VectorSubcoreMesh(core_axis_name="c", subcore_axis_name="s")

    @functools.partial(
        pl.kernel,
        out_type=jax.ShapeDtypeStruct((n_rows, 128), jnp.float32),
        mesh=mesh,
        scratch_types=[
            pltpu.VMEM((cpw, CHUNK), jnp.int32),
            pltpu.VMEM((2, CHUNK, 128), jnp.float32),
            pltpu.SemaphoreType.DMA,
            pltpu.SemaphoreType.DMA,
            pltpu.SemaphoreType.DMA,
            pltpu.SemaphoreType.DMA,
        ],
    )
    def k(idx_hbm, table_hbm, out_hbm, idx_v, rows_v, gs0, gs1, os0, os1):
        wid = lax.axis_index("s") * NC + lax.axis_index("c")
        gsems = (gs0, gs1)
        osems = (os0, os1)

        pltpu.sync_copy(idx_hbm.at[wid], idx_v)

        def gather(g, b):
            return pltpu.make_async_copy(
                table_hbm.at[idx_v.at[g]], rows_v.at[b], gsems[b])

        def scatter(g, b):
            return pltpu.make_async_copy(
                rows_v.at[b],
                out_hbm.at[pl.ds((wid * cpw + g) * CHUNK, CHUNK)],
                osems[b])

        gather(0, 0).start()

        @pl.loop(0, cpw, step=2)
        def _(i):
            for b in (0, 1):
                g = i + b
                @pl.when(g + 1 < cpw)
                def _():
                    @pl.when(g >= 1)
                    def _():
                        scatter(g - 1, 1 - b).wait()
                    gather(g + 1, 1 - b).start()

                gather(g, b).wait()
                scatter(g, b).start()

        scatter(cpw - 2, 0).wait()
        scatter(cpw - 1, 1).wait()

    return k(idx3d, table)


def _tc_finalize(g, par, batch, seq):
    def body(x_ref, par_ref, o_ref):
        xt = jnp.transpose(x_ref[...])
        p = par_ref[0]
        o_ref[...] = jnp.where(p == 1, xt[HID:], xt[:HID])[None]

    return pl.pallas_call(
        body,
        grid=(seq,),
        in_specs=[
            pl.BlockSpec((batch, 128), lambda i: (i, 0)),
            pl.BlockSpec((1, 1, batch), lambda i: (i, 0, 0)),
        ],
        out_specs=pl.BlockSpec((1, HID, batch), lambda i: (i, 0, 0)),
        out_shape=jax.ShapeDtypeStruct((seq, HID, batch), jnp.float32),
    )(g, par)


def kernel(sent1, sent2, embedding, W):
    batch, seq = sent1.shape
    proj = _tc_project_table(embedding.T, W)
    s1t = sent1.T.astype(jnp.int32)
    s2t = sent2.T.astype(jnp.int32)
    idx1 = jnp.where(s1t >= SPLIT, s1t - SPLIT, s1t).reshape(NW, -1, CHUNK)
    idx2 = jnp.where(s2t >= SPLIT, s2t - SPLIT, s2t).reshape(NW, -1, CHUNK)
    par1 = (s1t >= SPLIT).astype(jnp.int32).reshape(seq, 1, batch)
    par2 = (s2t >= SPLIT).astype(jnp.int32).reshape(seq, 1, batch)
    g1 = _sc_gather(idx1, proj)
    g2 = _sc_gather(idx2, proj)
    t1 = _tc_finalize(g1, par1, batch, seq)
    t2 = _tc_finalize(g2, par2, batch, seq)
    return (jnp.transpose(t1, (2, 0, 1)), jnp.transpose(t2, (2, 0, 1)))
